# baseline (speedup 1.0000x reference)
import functools

import jax
import jax.numpy as jnp
from jax import lax
from jax.experimental import pallas as pl
from jax.experimental.pallas import tpu as pltpu
from jax.experimental.pallas import tpu_sc as plsc

HIDDEN = 128
EPS = 1e-12

_A_ROWS = 13
_R_ROWS = 4
_A_PAD = 16
_R_PAD = 8
_C_PAD = 64

_NC, _NS, _L = 2, 16, 16
_NW = _NC * _NS
_CHUNK = 128


def _lane_sum(v):
    s = v[0]
    for i in range(1, _L):
        s = s + v[i]
    return s


def _rsqrt(x):
    bits = lax.bitcast_convert_type(x, jnp.int32)
    y = lax.bitcast_convert_type(
        jnp.int32(0x5F3759DF) - lax.shift_right_logical(bits, 1), jnp.float32)
    for _ in range(3):
        y = y * (1.5 - 0.5 * x * y * y)
    return y


def _gather_body(n_rows_w, a_hbm, r_hbm, g_hbm, b_hbm, i_hbm, out_hbm,
                 tab_s, a_v, r_v, g_v, b_v, row_v, idx_v, rows_v, gsem, osem):
    sid = lax.axis_index("s")
    wid = sid * _NC + lax.axis_index("c")
    base = wid * n_rows_w

    @pl.when(sid < _A_ROWS)
    def _():
        pltpu.sync_copy(a_hbm.at[sid], a_v)
        pltpu.sync_copy(r_hbm, r_v)
        pltpu.sync_copy(g_hbm, g_v)
        pltpu.sync_copy(b_hbm, b_v)
        nv = HIDDEN // _L
        for r in range(_R_ROWS):
            xs = [a_v[pl.ds(j * _L, _L)] + r_v[r, pl.ds(j * _L, _L)]
                  for j in range(nv)]
            s1 = xs[0]
            for j in range(1, nv):
                s1 = s1 + xs[j]
            mean = _lane_sum(s1) * (1.0 / HIDDEN)
            ds = [x - mean for x in xs]
            s2 = ds[0] * ds[0]
            for j in range(1, nv):
                s2 = s2 + ds[j] * ds[j]
            var = _lane_sum(s2) * (1.0 / HIDDEN)
            inv = _rsqrt(var + EPS)
            for j in range(nv):
                sl = pl.ds(j * _L, _L)
                row_v[r, sl] = ds[j] * inv * g_v[sl] + b_v[sl]
        pltpu.sync_copy(row_v, tab_s.at[pl.ds(sid * _R_ROWS, _R_ROWS)])

    pltpu.sync_copy(i_hbm.at[pl.ds(base, n_rows_w)], idx_v)
    plsc.subcore_barrier()

    nbuf, grows = rows_v.shape[0], rows_v.shape[1]
    g = grows // _CHUNK
    n_groups = n_rows_w // g
    n_outer = n_groups // nbuf

    def gather_copy(chunk, b, h):
        return pltpu.make_async_copy(
            tab_s.at[idx_v.at[chunk]],
            rows_v.at[b, pl.ds(h * _CHUNK, _CHUNK)], gsem)

    def out_copy(grp, b):
        return pltpu.make_async_copy(
            rows_v.at[b],
            out_hbm.at[pl.ds((base + grp * g) * _CHUNK, grows)], osem)

    for b in range(nbuf):
        for h in range(g):
            gather_copy(b * g + h, b, h).start()

    def outer(o, _):
        for b in range(nbuf):
            grp = o * nbuf + b
            for h in range(g):
                gather_copy(grp * g + h, b, h).wait()
            out_copy(grp, b).start()

        @pl.when(o + 1 < n_outer)
        def _():
            for b in range(nbuf):
                grp = o * nbuf + b
                out_copy(grp, b).wait()
                for h in range(g):
                    gather_copy((grp + nbuf) * g + h, b, h).start()
        return 0

    lax.fori_loop(0, n_outer, outer, 0)

    for b in range(nbuf):
        out_copy((n_outer - 1) * nbuf + b, b).wait()


def _gather(angle_table, rotation_table, ln_gamma, ln_beta, idx_flat, n):
    n_rows = n // _CHUNK
    n_rows_w = n_rows // _NW
    nbuf, g = 2, 2
    mesh = plsc.VectorSubcoreMesh(core_axis_name="c", subcore_axis_name="s")
    return pl.kernel(
        functools.partial(_gather_body, n_rows_w),
        out_type=jax.ShapeDtypeStruct((n, HIDDEN), jnp.float32),
        mesh=mesh,
        scratch_types=[
            pltpu.VMEM_SHARED((_C_PAD, HIDDEN), jnp.float32),
            pltpu.VMEM((HIDDEN,), jnp.float32),
            pltpu.VMEM((_R_ROWS, HIDDEN), jnp.float32),
            pltpu.VMEM((HIDDEN,), jnp.float32),
            pltpu.VMEM((HIDDEN,), jnp.float32),
            pltpu.VMEM((_R_ROWS, HIDDEN), jnp.float32),
            pltpu.VMEM((n_rows_w, _CHUNK), jnp.int32),
            pltpu.VMEM((nbuf, g * _CHUNK, HIDDEN), jnp.float32),
            pltpu.SemaphoreType.DMA,
            pltpu.SemaphoreType.DMA,
        ],
    )(angle_table, rotation_table, ln_gamma, ln_beta,
      idx_flat.reshape(n_rows, _CHUNK))


def kernel(input_angle, input_rotation, angle_table, rotation_table,
           ln_gamma, ln_beta):
    b, t = input_angle.shape
    n = input_angle.size
    idx_flat = (input_angle.astype(jnp.int32) * _R_ROWS
                + input_rotation.astype(jnp.int32)).T.reshape(-1)
    out = _gather(angle_table, rotation_table, ln_gamma, ln_beta, idx_flat, n)
    return out.reshape(t, b, HIDDEN).transpose(1, 0, 2)


# speedup vs baseline: 1.0121x; 1.0121x over previous
import functools

import jax
import jax.numpy as jnp
from jax import lax
from jax.experimental import pallas as pl
from jax.experimental.pallas import tpu as pltpu
from jax.experimental.pallas import tpu_sc as plsc

HIDDEN = 128
EPS = 1e-12

_A_ROWS = 13
_R_ROWS = 4
_A_PAD = 16
_R_PAD = 8
_C_PAD = 64

_NC, _NS, _L = 2, 16, 16
_NW = _NC * _NS
_CHUNK = 128


def _lane_sum(v):
    s = v[0]
    for i in range(1, _L):
        s = s + v[i]
    return s


def _rsqrt(x):
    bits = lax.bitcast_convert_type(x, jnp.int32)
    y = lax.bitcast_convert_type(
        jnp.int32(0x5F3759DF) - lax.shift_right_logical(bits, 1), jnp.float32)
    for _ in range(3):
        y = y * (1.5 - 0.5 * x * y * y)
    return y


def _gather_body(n_rows_w, a_hbm, r_hbm, g_hbm, b_hbm, i_hbm, out_hbm,
                 tab_s, a_v, r_v, g_v, b_v, row_v, idx_v, rows_v, gsem, osem):
    sid = lax.axis_index("s")
    wid = sid * _NC + lax.axis_index("c")
    base = wid * n_rows_w

    @pl.when(sid < _A_ROWS)
    def _():
        pltpu.sync_copy(a_hbm.at[sid], a_v)
        pltpu.sync_copy(r_hbm, r_v)
        pltpu.sync_copy(g_hbm, g_v)
        pltpu.sync_copy(b_hbm, b_v)
        nv = HIDDEN // _L
        for r in range(_R_ROWS):
            xs = [a_v[pl.ds(j * _L, _L)] + r_v[r, pl.ds(j * _L, _L)]
                  for j in range(nv)]
            s1 = xs[0]
            for j in range(1, nv):
                s1 = s1 + xs[j]
            mean = _lane_sum(s1) * (1.0 / HIDDEN)
            ds = [x - mean for x in xs]
            s2 = ds[0] * ds[0]
            for j in range(1, nv):
                s2 = s2 + ds[j] * ds[j]
            var = _lane_sum(s2) * (1.0 / HIDDEN)
            inv = _rsqrt(var + EPS)
            for j in range(nv):
                sl = pl.ds(j * _L, _L)
                row_v[r, sl] = ds[j] * inv * g_v[sl] + b_v[sl]
        pltpu.sync_copy(row_v, tab_s.at[pl.ds(sid * _R_ROWS, _R_ROWS)])

    pltpu.sync_copy(i_hbm.at[pl.ds(base, n_rows_w)], idx_v)
    plsc.subcore_barrier()

    nbuf, grows = rows_v.shape[0], rows_v.shape[1]
    g = grows // _CHUNK
    n_groups = n_rows_w // g
    n_outer = n_groups // nbuf

    def gather_copy(chunk, b, h):
        return pltpu.make_async_copy(
            tab_s.at[idx_v.at[chunk]],
            rows_v.at[b, pl.ds(h * _CHUNK, _CHUNK)], gsem)

    def out_copy(grp, b):
        return pltpu.make_async_copy(
            rows_v.at[b],
            out_hbm.at[pl.ds((base + grp * g) * _CHUNK, grows)], osem)

    for b in range(nbuf):
        for h in range(g):
            gather_copy(b * g + h, b, h).start()

    def outer(o, _):
        for b in range(nbuf):
            grp = o * nbuf + b
            for h in range(g):
                gather_copy(grp * g + h, b, h).wait()
            out_copy(grp, b).start()

        @pl.when(o + 1 < n_outer)
        def _():
            for b in range(nbuf):
                grp = o * nbuf + b
                out_copy(grp, b).wait()
                for h in range(g):
                    gather_copy((grp + nbuf) * g + h, b, h).start()
        return 0

    lax.fori_loop(0, n_outer, outer, 0)

    for b in range(nbuf):
        out_copy((n_outer - 1) * nbuf + b, b).wait()


def _gather(angle_table, rotation_table, ln_gamma, ln_beta, idx_flat, n):
    n_rows = n // _CHUNK
    n_rows_w = n_rows // _NW
    nbuf, g = 5, 1
    mesh = plsc.VectorSubcoreMesh(core_axis_name="c", subcore_axis_name="s")
    return pl.kernel(
        functools.partial(_gather_body, n_rows_w),
        out_type=jax.ShapeDtypeStruct((n, HIDDEN), jnp.float32),
        mesh=mesh,
        scratch_types=[
            pltpu.VMEM_SHARED((_C_PAD, HIDDEN), jnp.float32),
            pltpu.VMEM((HIDDEN,), jnp.float32),
            pltpu.VMEM((_R_ROWS, HIDDEN), jnp.float32),
            pltpu.VMEM((HIDDEN,), jnp.float32),
            pltpu.VMEM((HIDDEN,), jnp.float32),
            pltpu.VMEM((_R_ROWS, HIDDEN), jnp.float32),
            pltpu.VMEM((n_rows_w, _CHUNK), jnp.int32),
            pltpu.VMEM((nbuf, g * _CHUNK, HIDDEN), jnp.float32),
            pltpu.SemaphoreType.DMA,
            pltpu.SemaphoreType.DMA,
        ],
    )(angle_table, rotation_table, ln_gamma, ln_beta,
      idx_flat.reshape(n_rows, _CHUNK))


def kernel(input_angle, input_rotation, angle_table, rotation_table,
           ln_gamma, ln_beta):
    b, t = input_angle.shape
    n = input_angle.size
    idx_flat = (input_angle.astype(jnp.int32) * _R_ROWS
                + input_rotation.astype(jnp.int32)).T.reshape(-1)
    out = _gather(angle_table, rotation_table, ln_gamma, ln_beta, idx_flat, n)
    return out.reshape(t, b, HIDDEN).transpose(1, 0, 2)


# speedup vs baseline: 1.0123x; 1.0002x over previous
import functools

import jax
import jax.numpy as jnp
from jax import lax
from jax.experimental import pallas as pl
from jax.experimental.pallas import tpu as pltpu
from jax.experimental.pallas import tpu_sc as plsc

HIDDEN = 128
EPS = 1e-12

_A_ROWS = 13
_R_ROWS = 4
_C_PAD = 64

_NC, _NS, _L = 2, 16, 16
_NW = _NC * _NS
_CHUNK = 128


def _lane_sum(v):
    s = v[0]
    for i in range(1, _L):
        s = s + v[i]
    return s


def _rsqrt(x):
    bits = lax.bitcast_convert_type(x, jnp.int32)
    y = lax.bitcast_convert_type(
        jnp.int32(0x5F3759DF) - lax.shift_right_logical(bits, 1), jnp.float32)
    for _ in range(3):
        y = y * (1.5 - 0.5 * x * y * y)
    return y


def _gather_body(n_rows_w, a_hbm, r_hbm, g_hbm, b_hbm, i_hbm, out_hbm,
                 tab_s, a_v, r_v, g_v, b_v, row_v, idx_v, rows_v, gsem, osem):
    sid = lax.axis_index("s")
    wid = sid * _NC + lax.axis_index("c")
    base = wid * n_rows_w

    @pl.when(sid < _A_ROWS)
    def _():
        pltpu.sync_copy(a_hbm.at[sid], a_v)
        pltpu.sync_copy(r_hbm, r_v)
        pltpu.sync_copy(g_hbm, g_v)
        pltpu.sync_copy(b_hbm, b_v)
        nv = HIDDEN // _L
        for r in range(_R_ROWS):
            xs = [a_v[pl.ds(j * _L, _L)] + r_v[r, pl.ds(j * _L, _L)]
                  for j in range(nv)]
            s1 = xs[0]
            for j in range(1, nv):
                s1 = s1 + xs[j]
            mean = _lane_sum(s1) * (1.0 / HIDDEN)
            ds = [x - mean for x in xs]
            s2 = ds[0] * ds[0]
            for j in range(1, nv):
                s2 = s2 + ds[j] * ds[j]
            var = _lane_sum(s2) * (1.0 / HIDDEN)
            inv = _rsqrt(var + EPS)
            for j in range(nv):
                sl = pl.ds(j * _L, _L)
                row_v[r, sl] = ds[j] * inv * g_v[sl] + b_v[sl]
        pltpu.sync_copy(row_v, tab_s.at[pl.ds(sid * _R_ROWS, _R_ROWS)])

    pltpu.sync_copy(i_hbm.at[pl.ds(base, n_rows_w)], idx_v)
    plsc.subcore_barrier()

    nbuf, grows = rows_v.shape[0], rows_v.shape[1]
    g = grows // _CHUNK
    n_groups = n_rows_w // g
    n_outer = n_groups // nbuf

    def gather_copy(chunk, b, h):
        return pltpu.make_async_copy(
            tab_s.at[idx_v.at[chunk]],
            rows_v.at[b, pl.ds(h * _CHUNK, _CHUNK)], gsem)

    def out_copy(grp, b):
        return pltpu.make_async_copy(
            rows_v.at[b],
            out_hbm.at[pl.ds((base + grp * g) * _CHUNK, grows)], osem)

    for b in range(nbuf):
        for h in range(g):
            gather_copy(b * g + h, b, h).start()

    def outer(o, _):
        for b in range(nbuf):
            grp = o * nbuf + b
            for h in range(g):
                gather_copy(grp * g + h, b, h).wait()
            out_copy(grp, b).start()

        @pl.when(o + 1 < n_outer)
        def _():
            for b in range(nbuf):
                grp = o * nbuf + b
                out_copy(grp, b).wait()
                for h in range(g):
                    gather_copy((grp + nbuf) * g + h, b, h).start()
        return 0

    lax.fori_loop(0, n_outer, outer, 0)

    for b in range(nbuf):
        out_copy((n_outer - 1) * nbuf + b, b).wait()


def _gather(angle_table, rotation_table, ln_gamma, ln_beta, idx_flat, n):
    n_rows = n // _CHUNK
    n_rows_w = n_rows // _NW
    nbuf, g = 5, 1
    mesh = plsc.VectorSubcoreMesh(core_axis_name="c", subcore_axis_name="s")
    return pl.kernel(
        functools.partial(_gather_body, n_rows_w),
        out_type=jax.ShapeDtypeStruct((n, HIDDEN), jnp.float32),
        mesh=mesh,
        scratch_types=[
            pltpu.VMEM_SHARED((_C_PAD, HIDDEN), jnp.float32),
            pltpu.VMEM((HIDDEN,), jnp.float32),
            pltpu.VMEM((_R_ROWS, HIDDEN), jnp.float32),
            pltpu.VMEM((HIDDEN,), jnp.float32),
            pltpu.VMEM((HIDDEN,), jnp.float32),
            pltpu.VMEM((_R_ROWS, HIDDEN), jnp.float32),
            pltpu.VMEM((n_rows_w, _CHUNK), jnp.int32),
            pltpu.VMEM((nbuf, g * _CHUNK, HIDDEN), jnp.float32),
            pltpu.SemaphoreType.DMA,
            pltpu.SemaphoreType.DMA,
        ],
    )(angle_table, rotation_table, ln_gamma, ln_beta,
      idx_flat.reshape(n_rows, _CHUNK))


def kernel(input_angle, input_rotation, angle_table, rotation_table,
           ln_gamma, ln_beta):
    b, t = input_angle.shape
    n = input_angle.size
    idx_flat = (input_angle.astype(jnp.int32) * _R_ROWS
                + input_rotation.astype(jnp.int32)).T.reshape(-1)
    out = _gather(angle_table, rotation_table, ln_gamma, ln_beta, idx_flat, n)
    return out.reshape(t, b, HIDDEN).transpose(1, 0, 2)


# speedup vs baseline: 1.0361x; 1.0235x over previous
import functools

import jax
import jax.numpy as jnp
from jax import lax
from jax.experimental import pallas as pl
from jax.experimental.pallas import tpu as pltpu
from jax.experimental.pallas import tpu_sc as plsc

HIDDEN = 128
EPS = 1e-12

_A_ROWS = 13
_R_ROWS = 4
_C_PAD = 64

_NC, _NS, _L = 2, 16, 16
_NW = _NC * _NS
_CHUNK = 128


def _lane_sum(v):
    s = v[0]
    for i in range(1, _L):
        s = s + v[i]
    return s


def _rsqrt(x):
    bits = lax.bitcast_convert_type(x, jnp.int32)
    y = lax.bitcast_convert_type(
        jnp.int32(0x5F3759DF) - lax.shift_right_logical(bits, 1), jnp.float32)
    for _ in range(3):
        y = y * (1.5 - 0.5 * x * y * y)
    return y


def _gather_body(n_rows_w, a_hbm, r_hbm, g_hbm, b_hbm, i_hbm, out_hbm,
                 tab_s, a_v, r_v, g_v, b_v, row_v, idx_v, rows_v, gsem, osem):
    sid = lax.axis_index("s")
    wid = sid * _NC + lax.axis_index("c")
    base = wid * n_rows_w

    pltpu.make_async_copy(i_hbm.at[pl.ds(base, n_rows_w)], idx_v, osem).start()

    @pl.when(sid < _A_ROWS)
    def _():
        pltpu.make_async_copy(a_hbm.at[sid], a_v, gsem).start()
        pltpu.make_async_copy(r_hbm, r_v, gsem).start()
        pltpu.make_async_copy(g_hbm, g_v, gsem).start()
        pltpu.make_async_copy(b_hbm, b_v, gsem).start()
        pltpu.make_async_copy(a_hbm.at[sid], a_v, gsem).wait()
        pltpu.make_async_copy(r_hbm, r_v, gsem).wait()
        pltpu.make_async_copy(g_hbm, g_v, gsem).wait()
        pltpu.make_async_copy(b_hbm, b_v, gsem).wait()
        nv = HIDDEN // _L
        for r in range(_R_ROWS):
            xs = [a_v[pl.ds(j * _L, _L)] + r_v[r, pl.ds(j * _L, _L)]
                  for j in range(nv)]
            s1 = xs[0]
            for j in range(1, nv):
                s1 = s1 + xs[j]
            mean = _lane_sum(s1) * (1.0 / HIDDEN)
            ds = [x - mean for x in xs]
            s2 = ds[0] * ds[0]
            for j in range(1, nv):
                s2 = s2 + ds[j] * ds[j]
            var = _lane_sum(s2) * (1.0 / HIDDEN)
            inv = _rsqrt(var + EPS)
            for j in range(nv):
                sl = pl.ds(j * _L, _L)
                row_v[r, sl] = ds[j] * inv * g_v[sl] + b_v[sl]
        pltpu.sync_copy(row_v, tab_s.at[pl.ds(sid * _R_ROWS, _R_ROWS)])

    pltpu.make_async_copy(i_hbm.at[pl.ds(base, n_rows_w)], idx_v, osem).wait()
    plsc.subcore_barrier()

    nbuf, grows = rows_v.shape[0], rows_v.shape[1]
    g = grows // _CHUNK
    n_groups = n_rows_w // g
    n_outer = n_groups // nbuf

    def gather_copy(chunk, b, h):
        return pltpu.make_async_copy(
            tab_s.at[idx_v.at[chunk]],
            rows_v.at[b, pl.ds(h * _CHUNK, _CHUNK)], gsem)

    def out_copy(grp, b):
        return pltpu.make_async_copy(
            rows_v.at[b],
            out_hbm.at[pl.ds((base + grp * g) * _CHUNK, grows)], osem)

    for b in range(nbuf):
        for h in range(g):
            gather_copy(b * g + h, b, h).start()

    def outer(o, _):
        for b in range(nbuf):
            grp = o * nbuf + b
            for h in range(g):
                gather_copy(grp * g + h, b, h).wait()
            out_copy(grp, b).start()

        @pl.when(o + 1 < n_outer)
        def _():
            for b in range(nbuf):
                grp = o * nbuf + b
                out_copy(grp, b).wait()
                for h in range(g):
                    gather_copy((grp + nbuf) * g + h, b, h).start()
        return 0

    lax.fori_loop(0, n_outer, outer, 0)

    for b in range(nbuf):
        out_copy((n_outer - 1) * nbuf + b, b).wait()


def _gather(angle_table, rotation_table, ln_gamma, ln_beta, idx_flat, n):
    n_rows = n // _CHUNK
    n_rows_w = n_rows // _NW
    nbuf, g = 5, 1
    mesh = plsc.VectorSubcoreMesh(core_axis_name="c", subcore_axis_name="s")
    return pl.kernel(
        functools.partial(_gather_body, n_rows_w),
        out_type=jax.ShapeDtypeStruct((n, HIDDEN), jnp.float32),
        mesh=mesh,
        scratch_types=[
            pltpu.VMEM_SHARED((_C_PAD, HIDDEN), jnp.float32),
            pltpu.VMEM((HIDDEN,), jnp.float32),
            pltpu.VMEM((_R_ROWS, HIDDEN), jnp.float32),
            pltpu.VMEM((HIDDEN,), jnp.float32),
            pltpu.VMEM((HIDDEN,), jnp.float32),
            pltpu.VMEM((_R_ROWS, HIDDEN), jnp.float32),
            pltpu.VMEM((n_rows_w, _CHUNK), jnp.int32),
            pltpu.VMEM((nbuf, g * _CHUNK, HIDDEN), jnp.float32),
            pltpu.SemaphoreType.DMA,
            pltpu.SemaphoreType.DMA,
        ],
    )(angle_table, rotation_table, ln_gamma, ln_beta,
      idx_flat.reshape(n_rows, _CHUNK))


def kernel(input_angle, input_rotation, angle_table, rotation_table,
           ln_gamma, ln_beta):
    b, t = input_angle.shape
    n = input_angle.size
    idx_flat = (input_angle.astype(jnp.int32) * _R_ROWS
                + input_rotation.astype(jnp.int32)).T.reshape(-1)
    out = _gather(angle_table, rotation_table, ln_gamma, ln_beta, idx_flat, n)
    return out.reshape(t, b, HIDDEN).transpose(1, 0, 2)
